# baseline (speedup 1.0000x reference)
import functools

import jax
import jax.numpy as jnp
from jax import lax
from jax.experimental import pallas as pl
from jax.experimental.pallas import tpu as pltpu
from jax.experimental.pallas import tpu_sc as plsc

N_NODES = 10000
FDIM = 128
NGRAPH = 64

NC = 2
NS = 16
NW = NC * NS
EDGE_BLOCK = 128
BLOCKS_PER_W = 80
PHASE_BLOCKS = 20
NBUF = 2
E_PAD = NW * BLOCKS_PER_W * EDGE_BLOCK
PAD_EDGES = 7680
ROWS_PER_S = 632
N_PAD = NS * ROWS_PER_S


def _edge_agg_body(h_hbm, srcb_hbm, dstb_hbm, out_hbm, agg_sh, src_v, dst_v,
                   rows_v, *sems):
  c = lax.axis_index("c")
  s = lax.axis_index("s")
  wid = c * NS + s

  def _zero_row(i, carry):
    for j in range(FDIM // 16):
      rows_v[0, i, pl.ds(j * 16, 16)] = jnp.zeros((16,), jnp.float32)
    return carry

  lax.fori_loop(0, EDGE_BLOCK, _zero_row, 0)
  base = s * ROWS_PER_S
  for t in range(ROWS_PER_S // EDGE_BLOCK):
    pltpu.sync_copy(rows_v.at[0],
                    agg_sh.at[pl.ds(base + t * EDGE_BLOCK, EDGE_BLOCK)])
  rem = ROWS_PER_S % EDGE_BLOCK
  if rem:
    pltpu.sync_copy(rows_v.at[0, pl.ds(0, rem)],
                    agg_sh.at[pl.ds(base + ROWS_PER_S - rem, rem)])
  plsc.subcore_barrier()

  gsems = sems[:NBUF]
  ssems = sems[NBUF:]

  def _wait_gather(j, b):
    pltpu.make_async_copy(h_hbm.at[src_v.at[j]], rows_v.at[b], gsems[b]).wait()

  def _wait_scatter(j, b):
    pltpu.make_async_copy(rows_v.at[b], agg_sh.at[dst_v.at[j]],
                          ssems[b]).wait()

  n_phases = BLOCKS_PER_W // PHASE_BLOCKS
  for p in range(n_phases):
    pltpu.sync_copy(srcb_hbm.at[wid * n_phases + p], src_v)
    pltpu.sync_copy(dstb_hbm.at[wid * n_phases + p], dst_v)
    pltpu.async_copy(h_hbm.at[src_v.at[0]], rows_v.at[0], gsems[0])
    _wait_gather(0, 0)
    pltpu.async_copy(h_hbm.at[src_v.at[1]], rows_v.at[1], gsems[1])
    pltpu.async_copy(rows_v.at[0], agg_sh.at[dst_v.at[0]], ssems[0],
                     add=True)

    def _steady(t, carry):
      for k in range(NBUF):
        j = t * NBUF + 1 + k
        b = (1 + k) % NBUF
        bo = k % NBUF
        _wait_scatter(j, bo)
        pltpu.async_copy(h_hbm.at[src_v.at[j + 1]], rows_v.at[bo], gsems[bo])
        _wait_gather(j, b)
        pltpu.async_copy(rows_v.at[b], agg_sh.at[dst_v.at[j]], ssems[b],
                         add=True)
      return carry

    lax.fori_loop(0, (PHASE_BLOCKS - 2) // NBUF, _steady, 0)
    j = PHASE_BLOCKS - 1
    b = j % NBUF
    bo = (j + 1) % NBUF
    _wait_scatter(j, bo)
    _wait_gather(j, b)
    pltpu.async_copy(rows_v.at[b], agg_sh.at[dst_v.at[j]], ssems[b],
                     add=True)
    _wait_scatter(0, b)
  plsc.subcore_barrier()

  pltpu.sync_copy(agg_sh.at[pl.ds(s * ROWS_PER_S, ROWS_PER_S)],
                  out_hbm.at[c, pl.ds(s * ROWS_PER_S, ROWS_PER_S)])


_edge_agg = functools.partial(
    pl.kernel,
    out_type=jax.ShapeDtypeStruct((NC, N_PAD, FDIM), jnp.float32),
    mesh=plsc.VectorSubcoreMesh(core_axis_name="c", subcore_axis_name="s",
                                num_cores=NC, num_subcores=NS),
    scratch_types=[
        pltpu.VMEM_SHARED((N_PAD, FDIM), jnp.float32),
        pltpu.VMEM((PHASE_BLOCKS, EDGE_BLOCK), jnp.int32),
        pltpu.VMEM((PHASE_BLOCKS, EDGE_BLOCK), jnp.int32),
        pltpu.VMEM((NBUF, EDGE_BLOCK, FDIM), jnp.float32),
    ] + [pltpu.SemaphoreType.DMA] * (2 * NBUF),
)(_edge_agg_body)


def _pad_corrected_z(h_ref, aggs_ref):
  return h_ref[...] + aggs_ref[0, :N_NODES, :] + aggs_ref[1, :N_NODES, :]


def _layer_body(h_ref, aggs_ref, w1_ref, b1_ref, g1_ref, be1_ref, w2_ref,
                b2_ref, out_ref):
  z = _pad_corrected_z(h_ref, aggs_ref)
  y = jnp.dot(z, w1_ref[...], preferred_element_type=jnp.float32) + b1_ref[...]
  m = jnp.mean(y, axis=0, keepdims=True)
  v = jnp.mean((y - m) * (y - m), axis=0, keepdims=True)
  yn = g1_ref[...] * (y - m) * lax.rsqrt(v + 1e-5) + be1_ref[...]
  z2 = jnp.maximum(yn, 0.0)
  h2 = jnp.dot(z2, w2_ref[...], preferred_element_type=jnp.float32) + b2_ref[...]
  out_ref[...] = jnp.maximum(h2, 0.0)


def _tc_layer(h, aggs, w1, b1, g1, be1, w2, b2):
  return pl.pallas_call(
      _layer_body,
      out_shape=jax.ShapeDtypeStruct((N_NODES, FDIM), jnp.float32),
  )(h, aggs, w1, b1, g1, be1, w2, b2)


def _last_body(h_ref, aggs_ref, batch_ref, w1_ref, b1_ref, g1_ref, be1_ref,
               w2_ref, b2_ref, mw1_ref, mb1_ref, mg_ref, mbe_ref, mw2_ref,
               mb2_ref, out_ref):
  z = _pad_corrected_z(h_ref, aggs_ref)
  y = jnp.dot(z, w1_ref[...], preferred_element_type=jnp.float32) + b1_ref[...]
  m = jnp.mean(y, axis=0, keepdims=True)
  v = jnp.mean((y - m) * (y - m), axis=0, keepdims=True)
  yn = g1_ref[...] * (y - m) * lax.rsqrt(v + 1e-5) + be1_ref[...]
  z2 = jnp.maximum(yn, 0.0)
  h2 = jnp.dot(z2, w2_ref[...], preferred_element_type=jnp.float32) + b2_ref[...]
  h2 = jnp.maximum(h2, 0.0)
  y2 = jnp.dot(h2, mw1_ref[...], preferred_element_type=jnp.float32) + mb1_ref[...]
  m2 = jnp.mean(y2, axis=0, keepdims=True)
  v2 = jnp.mean((y2 - m2) * (y2 - m2), axis=0, keepdims=True)
  yn2 = mg_ref[...] * (y2 - m2) * lax.rsqrt(v2 + 1e-5) + mbe_ref[...]
  node = (jnp.dot(jnp.maximum(yn2, 0.0), mw2_ref[...],
                  preferred_element_type=jnp.float32) + mb2_ref[...])
  gids = lax.broadcasted_iota(jnp.int32, (NGRAPH, N_NODES), 0)
  onehot = jnp.where(batch_ref[...] == gids, 1.0, 0.0)
  out_ref[...] = jnp.dot(onehot, node, preferred_element_type=jnp.float32)


def _tc_last(h, aggs, batch2d, w1, b1, g1, be1, w2, b2, mw1, mb1, mg, mbe,
             mw2, mb2):
  return pl.pallas_call(
      _last_body,
      out_shape=jax.ShapeDtypeStruct((NGRAPH, FDIM), jnp.float32),
  )(h, aggs, batch2d, w1, b1, g1, be1, w2, b2, mw1, mb1, mg, mbe, mw2, mb2)


def kernel(x, edge_index, batch, conv0_W1, conv0_b1, conv0_g1, conv0_be1,
           conv0_W2, conv0_b2, conv1_W1, conv1_b1, conv1_g1, conv1_be1,
           conv1_W2, conv1_b2, conv2_W1, conv2_b1, conv2_g1, conv2_be1,
           conv2_W2, conv2_b2, mlp_W1, mlp_b1, mlp_g, mlp_be, mlp_W2, mlp_b2):
  src = edge_index[0]
  dst = edge_index[1]
  e = src.shape[0]
  pad_src = jnp.zeros((E_PAD - e,), jnp.int32)
  pad_dst = N_NODES + (jnp.arange(E_PAD - e, dtype=jnp.int32)
                       % (N_PAD - N_NODES))
  nph = BLOCKS_PER_W // PHASE_BLOCKS
  srcb = jnp.concatenate([src, pad_src]).reshape(NW * nph, PHASE_BLOCKS,
                                                 EDGE_BLOCK)
  dstb = jnp.concatenate([dst, pad_dst]).reshape(NW * nph, PHASE_BLOCKS,
                                                 EDGE_BLOCK)
  batch2d = batch.reshape(1, N_NODES)

  def r2(v):
    return v.reshape(1, FDIM)

  h = x
  aggs = _edge_agg(h, srcb, dstb)
  h = _tc_layer(h, aggs, conv0_W1, r2(conv0_b1), r2(conv0_g1), r2(conv0_be1),
                conv0_W2, r2(conv0_b2))
  aggs = _edge_agg(h, srcb, dstb)
  h = _tc_layer(h, aggs, conv1_W1, r2(conv1_b1), r2(conv1_g1), r2(conv1_be1),
                conv1_W2, r2(conv1_b2))
  aggs = _edge_agg(h, srcb, dstb)
  return _tc_last(h, aggs, batch2d, conv2_W1, r2(conv2_b1), r2(conv2_g1),
                  r2(conv2_be1), conv2_W2, r2(conv2_b2), mlp_W1, r2(mlp_b1),
                  r2(mlp_g), r2(mlp_be), mlp_W2, r2(mlp_b2))


# speedup vs baseline: 2.8731x; 2.8731x over previous
import functools

import jax
import jax.numpy as jnp
from jax import lax
from jax.experimental import pallas as pl
from jax.experimental.pallas import tpu as pltpu
from jax.experimental.pallas import tpu_sc as plsc

N_NODES = 10000
FDIM = 128
NGRAPH = 64

NC = 2
NS = 16
FH = FDIM // NC
EDGE_BLOCK = 128
BLOCKS_PER_T = 160
PHASE_BLOCKS = 40
NBUF = 2
E_PAD = NS * BLOCKS_PER_T * EDGE_BLOCK
ROWS_PER_S = 632
N_PAD = NS * ROWS_PER_S
H_TAIL = N_NODES - 15 * ROWS_PER_S


def _edge_agg_body(hsplit_hbm, srcb_hbm, dstb_hbm, out_hbm, h_sh, agg_sh,
                   src_v, dst_v, rows_v, *sems):
  c = lax.axis_index("c")
  s = lax.axis_index("s")

  base = s * ROWS_PER_S

  @pl.when(s < NS - 1)
  def _stage_main():
    pltpu.sync_copy(hsplit_hbm.at[c, pl.ds(base, ROWS_PER_S)],
                    h_sh.at[pl.ds(base, ROWS_PER_S)])

  @pl.when(s == NS - 1)
  def _stage_tail():
    pltpu.sync_copy(hsplit_hbm.at[c, pl.ds((NS - 1) * ROWS_PER_S, H_TAIL)],
                    h_sh.at[pl.ds((NS - 1) * ROWS_PER_S, H_TAIL)])

  def _zero_row(i, carry):
    for j in range(FH // 16):
      rows_v[0, i, pl.ds(j * 16, 16)] = jnp.zeros((16,), jnp.float32)
    return carry

  lax.fori_loop(0, EDGE_BLOCK, _zero_row, 0)
  for t in range(ROWS_PER_S // EDGE_BLOCK):
    pltpu.sync_copy(rows_v.at[0],
                    agg_sh.at[pl.ds(base + t * EDGE_BLOCK, EDGE_BLOCK)])
  rem = ROWS_PER_S % EDGE_BLOCK
  if rem:
    pltpu.sync_copy(rows_v.at[0, pl.ds(0, rem)],
                    agg_sh.at[pl.ds(base + ROWS_PER_S - rem, rem)])
  plsc.subcore_barrier()

  gsems = sems[:NBUF]
  ssems = sems[NBUF:]

  def _wait_gather(j, b):
    pltpu.make_async_copy(h_sh.at[src_v.at[j]], rows_v.at[b], gsems[b]).wait()

  def _wait_scatter(j, b):
    pltpu.make_async_copy(rows_v.at[b], agg_sh.at[dst_v.at[j]],
                          ssems[b]).wait()

  n_phases = BLOCKS_PER_T // PHASE_BLOCKS
  for p in range(n_phases):
    pltpu.sync_copy(srcb_hbm.at[s * n_phases + p], src_v)
    pltpu.sync_copy(dstb_hbm.at[s * n_phases + p], dst_v)
    pltpu.async_copy(h_sh.at[src_v.at[0]], rows_v.at[0], gsems[0])
    _wait_gather(0, 0)
    pltpu.async_copy(h_sh.at[src_v.at[1]], rows_v.at[1], gsems[1])
    pltpu.async_copy(rows_v.at[0], agg_sh.at[dst_v.at[0]], ssems[0],
                     add=True)

    def _steady(t, carry):
      for k in range(NBUF):
        j = t * NBUF + 1 + k
        b = (1 + k) % NBUF
        bo = k % NBUF
        _wait_scatter(j, bo)
        pltpu.async_copy(h_sh.at[src_v.at[j + 1]], rows_v.at[bo], gsems[bo])
        _wait_gather(j, b)
        pltpu.async_copy(rows_v.at[b], agg_sh.at[dst_v.at[j]], ssems[b],
                         add=True)
      return carry

    lax.fori_loop(0, (PHASE_BLOCKS - 2) // NBUF, _steady, 0)
    j = PHASE_BLOCKS - 1
    b = j % NBUF
    bo = (j + 1) % NBUF
    _wait_scatter(j, bo)
    _wait_gather(j, b)
    pltpu.async_copy(rows_v.at[b], agg_sh.at[dst_v.at[j]], ssems[b],
                     add=True)
    _wait_scatter(0, b)
  plsc.subcore_barrier()

  pltpu.sync_copy(agg_sh.at[pl.ds(base, ROWS_PER_S)],
                  out_hbm.at[c, pl.ds(base, ROWS_PER_S)])


_edge_agg = functools.partial(
    pl.kernel,
    out_type=jax.ShapeDtypeStruct((NC, N_PAD, FH), jnp.float32),
    mesh=plsc.VectorSubcoreMesh(core_axis_name="c", subcore_axis_name="s",
                                num_cores=NC, num_subcores=NS),
    scratch_types=[
        pltpu.VMEM_SHARED((N_PAD, FH), jnp.float32),
        pltpu.VMEM_SHARED((N_PAD, FH), jnp.float32),
        pltpu.VMEM((PHASE_BLOCKS, EDGE_BLOCK), jnp.int32),
        pltpu.VMEM((PHASE_BLOCKS, EDGE_BLOCK), jnp.int32),
        pltpu.VMEM((NBUF, EDGE_BLOCK, FH), jnp.float32),
    ] + [pltpu.SemaphoreType.DMA] * (2 * NBUF),
)(_edge_agg_body)


def _gin_z(h_ref, aggs_ref):
  return jnp.concatenate(
      [h_ref[0] + aggs_ref[0, :N_NODES, :],
       h_ref[1] + aggs_ref[1, :N_NODES, :]], axis=1)


def _layer_body(h_ref, aggs_ref, w1_ref, b1_ref, g1_ref, be1_ref, w2_ref,
                b2_ref, out_ref):
  z = _gin_z(h_ref, aggs_ref)
  y = jnp.dot(z, w1_ref[...], preferred_element_type=jnp.float32) + b1_ref[...]
  m = jnp.mean(y, axis=0, keepdims=True)
  v = jnp.mean((y - m) * (y - m), axis=0, keepdims=True)
  yn = g1_ref[...] * (y - m) * lax.rsqrt(v + 1e-5) + be1_ref[...]
  z2 = jnp.maximum(yn, 0.0)
  h2 = jnp.dot(z2, w2_ref[...], preferred_element_type=jnp.float32) + b2_ref[...]
  h2 = jnp.maximum(h2, 0.0)
  out_ref[0] = h2[:, :FH]
  out_ref[1] = h2[:, FH:]


def _tc_layer(h, aggs, w1, b1, g1, be1, w2, b2):
  return pl.pallas_call(
      _layer_body,
      out_shape=jax.ShapeDtypeStruct((NC, N_NODES, FH), jnp.float32),
  )(h, aggs, w1, b1, g1, be1, w2, b2)


def _last_body(h_ref, aggs_ref, batch_ref, w1_ref, b1_ref, g1_ref, be1_ref,
               w2_ref, b2_ref, mw1_ref, mb1_ref, mg_ref, mbe_ref, mw2_ref,
               mb2_ref, out_ref):
  z = _gin_z(h_ref, aggs_ref)
  y = jnp.dot(z, w1_ref[...], preferred_element_type=jnp.float32) + b1_ref[...]
  m = jnp.mean(y, axis=0, keepdims=True)
  v = jnp.mean((y - m) * (y - m), axis=0, keepdims=True)
  yn = g1_ref[...] * (y - m) * lax.rsqrt(v + 1e-5) + be1_ref[...]
  z2 = jnp.maximum(yn, 0.0)
  h2 = jnp.dot(z2, w2_ref[...], preferred_element_type=jnp.float32) + b2_ref[...]
  h2 = jnp.maximum(h2, 0.0)
  y2 = jnp.dot(h2, mw1_ref[...], preferred_element_type=jnp.float32) + mb1_ref[...]
  m2 = jnp.mean(y2, axis=0, keepdims=True)
  v2 = jnp.mean((y2 - m2) * (y2 - m2), axis=0, keepdims=True)
  yn2 = mg_ref[...] * (y2 - m2) * lax.rsqrt(v2 + 1e-5) + mbe_ref[...]
  node = (jnp.dot(jnp.maximum(yn2, 0.0), mw2_ref[...],
                  preferred_element_type=jnp.float32) + mb2_ref[...])
  gids = lax.broadcasted_iota(jnp.int32, (NGRAPH, N_NODES), 0)
  onehot = jnp.where(batch_ref[...] == gids, 1.0, 0.0)
  out_ref[...] = jnp.dot(onehot, node, preferred_element_type=jnp.float32)


def _tc_last(h, aggs, batch2d, w1, b1, g1, be1, w2, b2, mw1, mb1, mg, mbe,
             mw2, mb2):
  return pl.pallas_call(
      _last_body,
      out_shape=jax.ShapeDtypeStruct((NGRAPH, FDIM), jnp.float32),
  )(h, aggs, batch2d, w1, b1, g1, be1, w2, b2, mw1, mb1, mg, mbe, mw2, mb2)


def kernel(x, edge_index, batch, conv0_W1, conv0_b1, conv0_g1, conv0_be1,
           conv0_W2, conv0_b2, conv1_W1, conv1_b1, conv1_g1, conv1_be1,
           conv1_W2, conv1_b2, conv2_W1, conv2_b1, conv2_g1, conv2_be1,
           conv2_W2, conv2_b2, mlp_W1, mlp_b1, mlp_g, mlp_be, mlp_W2, mlp_b2):
  src = edge_index[0]
  dst = edge_index[1]
  e = src.shape[0]
  pad_src = jnp.zeros((E_PAD - e,), jnp.int32)
  pad_dst = N_NODES + (jnp.arange(E_PAD - e, dtype=jnp.int32)
                       % (N_PAD - N_NODES))
  nph = BLOCKS_PER_T // PHASE_BLOCKS
  srcb = jnp.concatenate([src, pad_src]).reshape(NS * nph, PHASE_BLOCKS,
                                                 EDGE_BLOCK)
  dstb = jnp.concatenate([dst, pad_dst]).reshape(NS * nph, PHASE_BLOCKS,
                                                 EDGE_BLOCK)
  batch2d = batch.reshape(1, N_NODES)

  def r2(v):
    return v.reshape(1, FDIM)

  h = jnp.stack([x[:, :FH], x[:, FH:]])
  aggs = _edge_agg(h, srcb, dstb)
  h = _tc_layer(h, aggs, conv0_W1, r2(conv0_b1), r2(conv0_g1), r2(conv0_be1),
                conv0_W2, r2(conv0_b2))
  aggs = _edge_agg(h, srcb, dstb)
  h = _tc_layer(h, aggs, conv1_W1, r2(conv1_b1), r2(conv1_g1), r2(conv1_be1),
                conv1_W2, r2(conv1_b2))
  aggs = _edge_agg(h, srcb, dstb)
  return _tc_last(h, aggs, batch2d, conv2_W1, r2(conv2_b1), r2(conv2_g1),
                  r2(conv2_be1), conv2_W2, r2(conv2_b2), mlp_W1, r2(mlp_b1),
                  r2(mlp_g), r2(mlp_be), mlp_W2, r2(mlp_b2))
